# baseline (speedup 1.0000x reference)
import jax
import jax.numpy as jnp
from jax.experimental import pallas as pl
from jax.experimental.pallas import tpu as pltpu

_K = 32
_B = 4096
_NB = 25
_NPAD = _B * _NB
_NSL = 32
_SL = _B // _NSL
_NP = 5
_PW = _NP * _SL
_QB = 256
_NQB = 4
_BIG = 2**30
_IMAX = 2147483647
_TMASK = 2047


def _knn_kernel(points_ref, feat_ref, tgt_ref, out_ref, fa_ref, tq_ref):
    lane = jax.lax.broadcasted_iota(jnp.int32, (_QB, _PW), 1)
    kpos = jax.lax.broadcasted_iota(jnp.int32, (_QB, _K), 1)
    bcol = jax.lax.broadcasted_iota(jnp.int32, (_QB, 256), 1)
    inf = jnp.float32(jnp.inf)

    def init(b, _):
        f = feat_ref[b]
        fa_ref[b, 0:16, :] = f
        fa_ref[b, 16:17, :] = jnp.sum(f * f, axis=0, keepdims=True)
        fa_ref[b, 17:18, :] = jnp.ones((1, _B), jnp.float32)
        t = jnp.clip(tgt_ref[b], 0.0, 1.0)
        tq_ref[b] = jnp.round(t * 2047.0).astype(jnp.int32)
        return 0
    jax.lax.fori_loop(0, _NB, init, 0)

    def m22(A, B):
        a1, a2 = A
        b1, b2 = B
        o1 = jnp.minimum(a1, b1)
        o2 = jnp.minimum(jnp.maximum(a1, b1), jnp.minimum(a2, b2))
        return o1, o2

    for qb in range(_NQB):
        q = points_ref[qb * _QB:(qb + 1) * _QB, :]
        q2 = jnp.sum(q * q, axis=1, keepdims=True)
        qm2 = q * -2.0

        def p1(b, carry):
            bm, pool = carry
            qf = jnp.dot(qm2, fa_ref[b, 0:16, :],
                         preferred_element_type=jnp.float32)
            d2 = (qf + fa_ref[b, 16:17, :]) + q2
            u = jax.lax.bitcast_convert_type(d2, jnp.int32)
            key = (u & ~_TMASK) | tq_ref[b]

            sl = [key[:, i * _SL:(i + 1) * _SL] for i in range(_NSL)]
            l2 = [(jnp.minimum(sl[2 * i], sl[2 * i + 1]),
                   jnp.maximum(sl[2 * i], sl[2 * i + 1]))
                  for i in range(_NSL // 2)]
            l3 = [m22(l2[2 * i], l2[2 * i + 1]) for i in range(8)]
            l4 = [m22(l3[2 * i], l3[2 * i + 1]) for i in range(4)]
            k1, k2 = m22(m22(l4[0], l4[1]), m22(l4[2], l4[3]))

            for i in range(8):
                m = jnp.min(k1[:, i * 16:(i + 1) * 16], axis=1,
                            keepdims=True)
                bm = jnp.where(bcol == b * 8 + i, m, bm)

            p = [pool[:, i * _SL:(i + 1) * _SL] for i in range(_NP)]
            for start, kin in ((0, k1), (1, k2)):
                t = kin
                for j in range(start, _NP):
                    nj = jnp.minimum(p[j], t)
                    if j < _NP - 1:
                        t = jnp.maximum(p[j], t)
                    p[j] = nj
            return bm, jnp.concatenate(p, axis=1)

        bm, pool = jax.lax.fori_loop(
            0, _NB, p1,
            (jnp.full((_QB, 256), _IMAX, jnp.int32),
             jnp.full((_QB, _PW), _IMAX, jnp.int32)))

        def ext(_, carry):
            bmc, _v = carry
            v = jnp.min(bmc, axis=1, keepdims=True)
            am = jnp.min(jnp.where(bmc == v, bcol, _BIG), axis=1,
                         keepdims=True)
            return jnp.where(bcol == am, _IMAX, bmc), v

        _, tkey = jax.lax.fori_loop(
            0, _K, ext, (bm, jnp.zeros((_QB, 1), jnp.int32)))
        cap_t = tkey | _TMASK

        dm0 = jnp.where(pool <= cap_t, pool, _IMAX)
        v0 = jnp.min(dm0, axis=1, keepdims=True)

        def cond(st):
            v, _dm, _c, _R, _Rt = st
            return jnp.min(v) < _IMAX

        def body(st):
            v, dm, c, R, Rt = st
            hit = dm == v
            n = jnp.sum(hit.astype(jnp.int32), axis=1, keepdims=True)
            n = jnp.where(v < _IMAX, n, 0)
            dm = jnp.where(hit, _IMAX, dm)
            vd = jax.lax.bitcast_convert_type((v & ~_TMASK) | 1024,
                                              jnp.float32)
            vd = jnp.where(v == _IMAX, inf, vd)
            vd = jnp.maximum(vd, 0.0)
            tv = (v & _TMASK).astype(jnp.float32) * (1.0 / 2047.0)
            put = (kpos >= c) & (kpos < c + n)
            R = jnp.where(put, vd, R)
            Rt = jnp.where(put, tv, Rt)
            c = c + n
            vn = jnp.min(dm, axis=1, keepdims=True)
            vn = jnp.where(c < _K, vn, _IMAX)
            return vn, dm, c, R, Rt

        _, _, _, R, Rt = jax.lax.while_loop(
            cond, body,
            (v0, dm0, jnp.zeros((_QB, 1), jnp.int32),
             jnp.full((_QB, _K), inf, jnp.float32),
             jnp.zeros((_QB, _K), jnp.float32)))

        w = 1.0 / (R + 1e-4)
        num = jnp.sum(w * Rt, axis=1, keepdims=True)
        den = jnp.maximum(jnp.sum(w, axis=1, keepdims=True), 1e-9)
        out_ref[qb * _QB:(qb + 1) * _QB, :] = num / den


def kernel(points, historical_features, historical_targets):
    p = points.astype(jnp.float32)
    f = historical_features.astype(jnp.float32)
    t = historical_targets.astype(jnp.float32)
    n = f.shape[0]
    fp = jnp.pad(f, ((0, _NPAD - n), (0, 0)), constant_values=1e15)
    tp = jnp.pad(t, (0, _NPAD - n))
    f3 = fp.T.reshape(16, _NB, _B).transpose(1, 0, 2)
    t3 = tp.reshape(_NB, 1, _B)
    out = pl.pallas_call(
        _knn_kernel,
        out_shape=jax.ShapeDtypeStruct((_NQB * _QB, 1), jnp.float32),
        scratch_shapes=[pltpu.VMEM((_NB, 18, _B), jnp.float32),
                        pltpu.VMEM((_NB, 1, _B), jnp.int32)],
    )(p, f3, t3)
    return out.reshape(-1)
